# Initial kernel scaffold; baseline (speedup 1.0000x reference)
import jax
import jax.numpy as jnp
from jax.experimental import pallas as pl


def kernel(x, gaussian_means, gaussian_mats, subgaussian_frequency, subgaussian_offset, subgaussian_flat_top_power, subgaussian_width, subgaussian_rotation, colors):
    raise NotImplementedError("write your pallas kernel here")


# speedup vs baseline: 1.9524x; 1.9524x over previous
import jax
import jax.numpy as jnp
import numpy as np
from jax.experimental import pallas as pl
from jax.experimental.pallas import tpu as pltpu

N_CHANNELS = 3
BLOCK_N = 1024

_TWO_PI = 2.0 * np.pi


def _body(x_ref, pr_ref, col_ref, out_ref):
    x0 = x_ref[:, 0:1]
    x1 = x_ref[:, 1:2]

    meanx = pr_ref[0:1, :]
    meany = pr_ref[1:2, :]
    m00 = pr_ref[2:3, :]
    m01 = pr_ref[3:4, :]
    m10 = pr_ref[4:5, :]
    m11 = pr_ref[5:6, :]
    rot = pr_ref[6:7, :]
    freq = pr_ref[7:8, :]
    off = pr_ref[8:9, :]
    ftp = pr_ref[9:10, :]
    logw = pr_ref[10:11, :]

    c = jnp.cos(rot)
    s = jnp.sin(rot)
    inv_w2 = jnp.exp(-2.0 * logw)
    p = jnp.exp(ftp)

    relx = x0 - meanx
    rely = x1 - meany
    t0 = m00 * relx + m01 * rely
    t1 = m10 * relx + m11 * rely
    q = t0 * t0 + t1 * t1

    coord = relx * c + rely * s
    wave = jnp.sin(_TWO_PI * freq * coord + off)
    base = wave * wave * inv_w2 + 1e-12
    vals = jnp.exp(-0.5 * (q + jnp.exp(p * jnp.log(base))))

    out_ref[:, :] = jnp.dot(vals, col_ref[:, :],
                            preferred_element_type=jnp.float32)


@jax.jit
def kernel(x, gaussian_means, gaussian_mats, subgaussian_frequency,
           subgaussian_offset, subgaussian_flat_top_power,
           subgaussian_width, subgaussian_rotation, colors):
    n_pix = x.shape[0]
    w = gaussian_means.shape[0]

    params = jnp.concatenate([
        gaussian_means[:, 0][None, :],
        gaussian_means[:, 1][None, :],
        gaussian_mats[:, 0, 0][None, :],
        gaussian_mats[:, 0, 1][None, :],
        gaussian_mats[:, 1, 0][None, :],
        gaussian_mats[:, 1, 1][None, :],
        subgaussian_rotation.T,
        subgaussian_frequency.T,
        subgaussian_offset.T,
        subgaussian_flat_top_power.T,
        subgaussian_width.T,
        jnp.zeros((5, w), jnp.float32),
    ], axis=0)

    grid = (n_pix // BLOCK_N,)
    return pl.pallas_call(
        _body,
        grid=grid,
        in_specs=[
            pl.BlockSpec((BLOCK_N, 2), lambda i: (i, 0)),
            pl.BlockSpec((16, w), lambda i: (0, 0)),
            pl.BlockSpec((w, N_CHANNELS), lambda i: (0, 0)),
        ],
        out_specs=pl.BlockSpec((BLOCK_N, N_CHANNELS), lambda i: (i, 0)),
        out_shape=jax.ShapeDtypeStruct((n_pix, N_CHANNELS), jnp.float32),
        compiler_params=pltpu.CompilerParams(
            dimension_semantics=("parallel",),
        ),
    )(x, params, colors)


# speedup vs baseline: 4.0771x; 2.0883x over previous
import jax
import jax.numpy as jnp
import numpy as np
from jax.experimental import pallas as pl
from jax.experimental.pallas import tpu as pltpu

N_CHANNELS = 3
BLOCK_N = 1024

_RND = 12582912.0
_LN_HALF = float(np.log(0.5))
_SQRT_HALF = float(np.sqrt(0.5))
_COS_COEF = (1.0, -19.739208, 64.939384, -85.45664, 60.24202,
             -26.404266, 7.799566, -1.4530462)


def _body(x_ref, pr_ref, col_ref, out_ref):
    x0 = x_ref[:, 0:1]
    x1 = x_ref[:, 1:2]

    meanx = pr_ref[0:1, :]
    meany = pr_ref[1:2, :]
    m00 = pr_ref[2:3, :]
    m01 = pr_ref[3:4, :]
    m10 = pr_ref[4:5, :]
    m11 = pr_ref[5:6, :]
    rot = pr_ref[6:7, :]
    freq = pr_ref[7:8, :]
    off = pr_ref[8:9, :]
    ftp = pr_ref[9:10, :]
    logw = pr_ref[10:11, :]

    c = jnp.cos(rot)
    s = jnp.sin(rot)
    a0 = _SQRT_HALF * m00
    b0 = _SQRT_HALF * m01
    c0 = -(a0 * meanx + b0 * meany)
    a1 = _SQRT_HALF * m10
    b1 = _SQRT_HALF * m11
    c1 = -(a1 * meanx + b1 * meany)
    f2 = 2.0 * freq
    fa = f2 * c
    fb = f2 * s
    fc = off * (1.0 / np.pi) - (fa * meanx + fb * meany)
    half_inv_w2 = 0.5 * jnp.exp(-2.0 * logw)
    hw_eps = half_inv_w2 + 1e-12
    p = jnp.exp(ftp)

    u0 = a0 * x0 + (b0 * x1 + c0)
    u1 = a1 * x0 + (b1 * x1 + c1)
    v = fa * x0 + (fb * x1 + fc)

    r = jax.lax.round(v, jax.lax.RoundingMethod.TO_NEAREST_EVEN)
    sf = v - r
    t = sf * sf
    ct = jnp.float32(_COS_COEF[7])
    for k in (6, 5, 4, 3, 2, 1, 0):
        ct = ct * t + jnp.float32(_COS_COEF[k])
    base = jnp.maximum(hw_eps - ct * half_inv_w2, 1e-12)
    inner = p * jnp.log(base) + _LN_HALF
    arg = u0 * u0 + (u1 * u1 + jnp.exp(inner))
    vals = jnp.exp(-arg)

    out_ref[:, :] = jnp.dot(vals, col_ref[:, :],
                            preferred_element_type=jnp.float32)


@jax.jit
def kernel(x, gaussian_means, gaussian_mats, subgaussian_frequency,
           subgaussian_offset, subgaussian_flat_top_power,
           subgaussian_width, subgaussian_rotation, colors):
    n_pix = x.shape[0]
    w = gaussian_means.shape[0]

    params = jnp.concatenate([
        gaussian_means[:, 0][None, :],
        gaussian_means[:, 1][None, :],
        gaussian_mats[:, 0, 0][None, :],
        gaussian_mats[:, 0, 1][None, :],
        gaussian_mats[:, 1, 0][None, :],
        gaussian_mats[:, 1, 1][None, :],
        subgaussian_rotation.T,
        subgaussian_frequency.T,
        subgaussian_offset.T,
        subgaussian_flat_top_power.T,
        subgaussian_width.T,
        jnp.zeros((5, w), jnp.float32),
    ], axis=0)

    grid = (n_pix // BLOCK_N,)
    return pl.pallas_call(
        _body,
        grid=grid,
        in_specs=[
            pl.BlockSpec((BLOCK_N, 2), lambda i: (i, 0)),
            pl.BlockSpec((16, w), lambda i: (0, 0)),
            pl.BlockSpec((w, N_CHANNELS), lambda i: (0, 0)),
        ],
        out_specs=pl.BlockSpec((BLOCK_N, N_CHANNELS), lambda i: (i, 0)),
        out_shape=jax.ShapeDtypeStruct((n_pix, N_CHANNELS), jnp.float32),
        compiler_params=pltpu.CompilerParams(
            dimension_semantics=("parallel",),
        ),
    )(x, params, colors)
